# baseline (speedup 1.0000x reference)
import functools

import jax
import jax.numpy as jnp
from jax import lax
from jax.experimental import pallas as pl
from jax.experimental.pallas import tpu as pltpu
from jax.experimental.pallas import tpu_sc as plsc

_N = 10000
_D = 128
_F = 4
_G = 128
_C = 10
_BM = 256
_HALF = 5120
_NBLK = _HALF // _BM

_NW = 32
_CHUNK = _HALF // _NW
_NVEC = _CHUNK // 16
_GP = 136
_BUF = _F * _GP

_B_ROWS = _N - _HALF
_B_FULLW = _B_ROWS // _CHUNK
_B_TAILROWS = _B_ROWS - _B_FULLW * _CHUNK
_B_TAILVEC = _B_TAILROWS // 16


def _stream_a(filtre_ref, x_ref, w_ref, b_ref, ht_ref, y_ref, y_scr):
    i = pl.program_id(0)

    @pl.when(i == 0)
    def _init():
        y = jnp.dot(x_ref[...], w_ref[...],
                    preferred_element_type=jnp.float32).astype(jnp.bfloat16)
        y_scr[...] = y
        y_ref[...] = y

    fblk = filtre_ref[...].astype(jnp.bfloat16)
    h = jnp.dot(fblk, y_scr[...],
                preferred_element_type=jnp.float32)
    h = jnp.maximum(h + b_ref[...], 0.0)
    ht_ref[...] = h.T


def _stream_b(filtre_ref, y_ref, b_ref, ht_ref):
    fblk = filtre_ref[...].astype(jnp.bfloat16)
    h = jnp.dot(fblk, y_ref[...],
                preferred_element_type=jnp.float32)
    h = jnp.maximum(h + b_ref[...], 0.0)
    ht_ref[...] = h.T


def _lane_shuffle(x, src):
    dnums = lax.GatherDimensionNumbers(
        offset_dims=(), collapsed_slice_dims=(0,), start_index_map=(0,))
    return lax.gather(x, src[:, None], dnums, slice_sizes=(1,),
                      mode=lax.GatherScatterMode.PROMISE_IN_BOUNDS)


def _seg_pool_body(h_v, ind_v, buf_v, nvec, lane, nxt, shift_src):
    for j in range(nvec):
        idx = ind_v[pl.ds(j * 16, 16)]
        conds = [idx == _lane_shuffle(idx, s) for s in shift_src]
        idx_n = _lane_shuffle(idx, nxt)
        last = (idx != idx_n) | (lane == 15)
        for f in range(_F):
            v = h_v[pl.ds(f * _CHUNK + j * 16, 16)]
            for s, cond in zip(shift_src, conds):
                v_s = _lane_shuffle(v, s)
                v = jnp.where(cond, jnp.maximum(v, v_s), v)
            flat = idx + f * _GP
            old = plsc.load_gather(buf_v, [flat], mask=last)
            merged = jnp.maximum(v, old)
            plsc.store_scatter(buf_v, [flat], merged, mask=last)


def _make_sc_pool(ind_base, full_workers, tail_rows, tail_vec):
    def _sc_pool(h_hbm, ind_hbm, out_hbm, h_v, ind_v, buf_v, sem0, sem1):
        wid = lax.axis_index("s") * 2 + lax.axis_index("c")
        base = wid * _CHUNK

        zero16 = jnp.zeros((16,), jnp.float32)
        for j in range(_BUF // 16):
            buf_v[pl.ds(j * 16, 16)] = zero16

        lane = lax.iota(jnp.int32, 16)
        nxt = jnp.minimum(lane + 1, 15)
        shift_src = [jnp.maximum(lane - st, 0) for st in (1, 2, 4, 8)]

        @pl.when(wid < full_workers)
        def _full():
            c0 = pltpu.async_copy(ind_hbm.at[pl.ds(ind_base + base, _CHUNK)],
                                  ind_v, sem0)
            hc = [pltpu.async_copy(h_hbm.at[pl.ds(f * _HALF + base, _CHUNK)],
                                   h_v.at[pl.ds(f * _CHUNK, _CHUNK)], sem1)
                  for f in range(_F)]
            c0.wait()
            for c in hc:
                c.wait()
            _seg_pool_body(h_v, ind_v, buf_v, _NVEC, lane, nxt, shift_src)

        if tail_rows:
            @pl.when(wid == full_workers)
            def _tail():
                tbase = full_workers * _CHUNK
                c0 = pltpu.async_copy(
                    ind_hbm.at[pl.ds(ind_base + tbase, tail_rows)],
                    ind_v.at[pl.ds(0, tail_rows)], sem0)
                hc = [pltpu.async_copy(
                    h_hbm.at[pl.ds(f * _HALF + tbase, tail_rows)],
                    h_v.at[pl.ds(f * _CHUNK, tail_rows)], sem1)
                    for f in range(_F)]
                c0.wait()
                for c in hc:
                    c.wait()
                _seg_pool_body(h_v, ind_v, buf_v, tail_vec, lane, nxt,
                               shift_src)

        pltpu.sync_copy(buf_v, out_hbm.at[wid])
    return _sc_pool


def _finalize(pa_ref, pb_ref, wc_ref, bc_ref, out_ref):
    red = jnp.maximum(jnp.max(pa_ref[...], axis=0, keepdims=True),
                      jnp.max(pb_ref[...], axis=0, keepdims=True))
    pooled = jnp.concatenate(
        [red[:, f * _GP:f * _GP + _G] for f in range(_F)], axis=0)
    logits = jax.lax.dot_general(
        pooled, wc_ref[...], (((0,), (0,)), ((), ())),
        preferred_element_type=jnp.float32) + bc_ref[...]
    m = jnp.max(logits, axis=1, keepdims=True)
    e = jnp.exp(logits - m)
    out_ref[...] = e / jnp.sum(e, axis=1, keepdims=True)


def _sc_call(body):
    return functools.partial(
        pl.kernel,
        mesh=plsc.VectorSubcoreMesh(core_axis_name="c", subcore_axis_name="s"),
        out_type=jax.ShapeDtypeStruct((_NW, _BUF), jnp.float32),
        scratch_types=[
            pltpu.VMEM((_F * _CHUNK,), jnp.float32),
            pltpu.VMEM((_CHUNK,), jnp.int32),
            pltpu.VMEM((_BUF,), jnp.float32),
            pltpu.SemaphoreType.DMA,
            pltpu.SemaphoreType.DMA,
        ],
        compiler_params=pltpu.CompilerParams(needs_layout_passes=False),
    )(body)


def kernel(filtre, X, node_indicator, W, b, Wc, bc):
    b2d = b.reshape(1, _F)
    bc2d = bc.reshape(1, _C)
    ind32 = node_indicator.astype(jnp.int32)

    ht_a, y = pl.pallas_call(
        _stream_a,
        grid=(_NBLK,),
        in_specs=[
            pl.BlockSpec((_BM, _N), lambda i: (i, 0)),
            pl.BlockSpec((_N, _D), lambda i: (0, 0)),
            pl.BlockSpec((_D, _F), lambda i: (0, 0)),
            pl.BlockSpec((1, _F), lambda i: (0, 0)),
        ],
        out_specs=[
            pl.BlockSpec((_F, _BM), lambda i: (0, i)),
            pl.BlockSpec((_N, _F), lambda i: (0, 0)),
        ],
        out_shape=[
            jax.ShapeDtypeStruct((_F, _HALF), jnp.float32),
            jax.ShapeDtypeStruct((_N, _F), jnp.bfloat16),
        ],
        scratch_shapes=[pltpu.VMEM((_N, _F), jnp.bfloat16)],
        compiler_params=pltpu.CompilerParams(
            dimension_semantics=("arbitrary",),
            vmem_limit_bytes=100 * 1024 * 1024,
        ),
    )(filtre, X, W, b2d)

    part_a = _sc_call(_make_sc_pool(0, _NW, 0, 0))(
        ht_a.reshape(_F * _HALF), ind32)

    nblk_b = _NBLK
    ht_b = pl.pallas_call(
        _stream_b,
        grid=(nblk_b,),
        in_specs=[
            pl.BlockSpec((_BM, _N), lambda i: (i + _NBLK, 0)),
            pl.BlockSpec((_N, _F), lambda i: (0, 0)),
            pl.BlockSpec((1, _F), lambda i: (0, 0)),
        ],
        out_specs=pl.BlockSpec((_F, _BM), lambda i: (0, i)),
        out_shape=jax.ShapeDtypeStruct((_F, _HALF), jnp.float32),
        compiler_params=pltpu.CompilerParams(
            dimension_semantics=("arbitrary",),
            vmem_limit_bytes=100 * 1024 * 1024,
        ),
    )(filtre, y, b2d)

    part_b = _sc_call(_make_sc_pool(_HALF, _B_FULLW, _B_TAILROWS, _B_TAILVEC))(
        ht_b.reshape(_F * _HALF), ind32)

    return pl.pallas_call(
        _finalize,
        in_specs=[
            pl.BlockSpec((_NW, _BUF), lambda: (0, 0)),
            pl.BlockSpec((_NW, _BUF), lambda: (0, 0)),
            pl.BlockSpec((_F, _C), lambda: (0, 0)),
            pl.BlockSpec((1, _C), lambda: (0, 0)),
        ],
        out_specs=pl.BlockSpec((_G, _C), lambda: (0, 0)),
        out_shape=jax.ShapeDtypeStruct((_G, _C), jnp.float32),
    )(part_a, part_b, Wc, bc2d)


# speedup vs baseline: 1.0578x; 1.0578x over previous
import functools

import jax
import jax.numpy as jnp
from jax import lax
from jax.experimental import pallas as pl
from jax.experimental.pallas import tpu as pltpu
from jax.experimental.pallas import tpu_sc as plsc

_N = 10000
_D = 128
_F = 4
_G = 128
_C = 10
_BM = 256
_NBLK = 40

_NW = 32
_NP = 10240
_CHUNK = _NP // _NW
_NVEC = _CHUNK // 16
_GP = 136
_BUF = _F * _GP


def _stream(filtre_ref, x_ref, w_ref, b_ref, h_ref, y_scr):
    i = pl.program_id(0)

    @pl.when(i == 0)
    def _init():
        y_scr[...] = jnp.dot(x_ref[...], w_ref[...],
                             preferred_element_type=jnp.float32
                             ).astype(jnp.bfloat16)

    fblk = filtre_ref[...].astype(jnp.bfloat16)
    h = jnp.dot(fblk, y_scr[...],
                preferred_element_type=jnp.float32)
    h = jnp.maximum(h + b_ref[...], 0.0)
    h_ref[...] = h.T


def _lane_shuffle(x, src):
    dnums = lax.GatherDimensionNumbers(
        offset_dims=(), collapsed_slice_dims=(0,), start_index_map=(0,))
    return lax.gather(x, src[:, None], dnums, slice_sizes=(1,),
                      mode=lax.GatherScatterMode.PROMISE_IN_BOUNDS)


_TAILW = _NW - 1
_TAILROWS = _N - _TAILW * _CHUNK
_TAILVEC = _TAILROWS // 16


def _sc_pool(h_hbm, ind_hbm, out_hbm, h_v, ind_v, buf_v, sem0, sem1):
    wid = lax.axis_index("s") * 2 + lax.axis_index("c")
    base = wid * _CHUNK

    zero16 = jnp.zeros((16,), jnp.float32)
    for j in range(_BUF // 16):
        buf_v[pl.ds(j * 16, 16)] = zero16

    lane = lax.iota(jnp.int32, 16)
    nxt = jnp.minimum(lane + 1, 15)
    shift_src = [jnp.maximum(lane - st, 0) for st in (1, 2, 4, 8)]

    def run(nvec):
        for j in range(nvec):
            idx = ind_v[pl.ds(j * 16, 16)]
            conds = [idx == _lane_shuffle(idx, s) for s in shift_src]
            idx_n = _lane_shuffle(idx, nxt)
            last = (idx != idx_n) | (lane == 15)
            for f in range(_F):
                v = h_v[pl.ds(f * _CHUNK + j * 16, 16)]
                for s, cond in zip(shift_src, conds):
                    v_s = _lane_shuffle(v, s)
                    v = jnp.where(cond, jnp.maximum(v, v_s), v)
                flat = idx + f * _GP
                old = plsc.load_gather(buf_v, [flat], mask=last)
                merged = jnp.maximum(v, old)
                plsc.store_scatter(buf_v, [flat], merged, mask=last)

    @pl.when(wid < _TAILW)
    def _full():
        c0 = pltpu.async_copy(ind_hbm.at[pl.ds(base, _CHUNK)], ind_v, sem0)
        hc = [pltpu.async_copy(h_hbm.at[pl.ds(f * _NP + base, _CHUNK)],
                               h_v.at[pl.ds(f * _CHUNK, _CHUNK)], sem1)
              for f in range(_F)]
        c0.wait()
        for c in hc:
            c.wait()
        run(_NVEC)

    @pl.when(wid == _TAILW)
    def _tail():
        tbase = _TAILW * _CHUNK
        c0 = pltpu.async_copy(ind_hbm.at[pl.ds(tbase, _TAILROWS)],
                              ind_v.at[pl.ds(0, _TAILROWS)], sem0)
        hc = [pltpu.async_copy(h_hbm.at[pl.ds(f * _NP + tbase, _TAILROWS)],
                               h_v.at[pl.ds(f * _CHUNK, _TAILROWS)], sem1)
              for f in range(_F)]
        c0.wait()
        for c in hc:
            c.wait()
        run(_TAILVEC)

    pltpu.sync_copy(buf_v, out_hbm.at[wid])


def _finalize(part_ref, wc_ref, bc_ref, out_ref):
    red = jnp.max(part_ref[...], axis=0, keepdims=True)
    pooled = jnp.concatenate(
        [red[:, f * _GP:f * _GP + _G] for f in range(_F)], axis=0)
    logits = jax.lax.dot_general(
        pooled, wc_ref[...], (((0,), (0,)), ((), ())),
        preferred_element_type=jnp.float32) + bc_ref[...]
    m = jnp.max(logits, axis=1, keepdims=True)
    e = jnp.exp(logits - m)
    out_ref[...] = e / jnp.sum(e, axis=1, keepdims=True)


def kernel(filtre, X, node_indicator, W, b, Wc, bc):
    b2d = b.reshape(1, _F)
    bc2d = bc.reshape(1, _C)

    h = pl.pallas_call(
        _stream,
        grid=(_NBLK,),
        in_specs=[
            pl.BlockSpec((_BM, _N), lambda i: (i, 0)),
            pl.BlockSpec((_N, _D), lambda i: (0, 0)),
            pl.BlockSpec((_D, _F), lambda i: (0, 0)),
            pl.BlockSpec((1, _F), lambda i: (0, 0)),
        ],
        out_specs=pl.BlockSpec((_F, _BM), lambda i: (0, i)),
        out_shape=jax.ShapeDtypeStruct((_F, _NP), jnp.float32),
        scratch_shapes=[pltpu.VMEM((_N, _F), jnp.bfloat16)],
        compiler_params=pltpu.CompilerParams(
            dimension_semantics=("arbitrary",),
            vmem_limit_bytes=100 * 1024 * 1024,
        ),
    )(filtre, X, W, b2d)

    h_flat = h.reshape(_F * _NP)
    ind32 = node_indicator.astype(jnp.int32)

    sc_pool = functools.partial(
        pl.kernel,
        mesh=plsc.VectorSubcoreMesh(core_axis_name="c", subcore_axis_name="s"),
        out_type=jax.ShapeDtypeStruct((_NW, _BUF), jnp.float32),
        scratch_types=[
            pltpu.VMEM((_F * _CHUNK,), jnp.float32),
            pltpu.VMEM((_CHUNK,), jnp.int32),
            pltpu.VMEM((_BUF,), jnp.float32),
            pltpu.SemaphoreType.DMA,
            pltpu.SemaphoreType.DMA,
        ],
        compiler_params=pltpu.CompilerParams(needs_layout_passes=False),
    )(_sc_pool)
    part = sc_pool(h_flat, ind32)

    return pl.pallas_call(
        _finalize,
        in_specs=[
            pl.BlockSpec((_NW, _BUF), lambda: (0, 0)),
            pl.BlockSpec((_F, _C), lambda: (0, 0)),
            pl.BlockSpec((1, _C), lambda: (0, 0)),
        ],
        out_specs=pl.BlockSpec((_G, _C), lambda: (0, 0)),
        out_shape=jax.ShapeDtypeStruct((_G, _C), jnp.float32),
    )(part, Wc, bc2d)
